# baseline (speedup 1.0000x reference)
import functools

import jax
import jax.numpy as jnp
from jax import lax
from jax.experimental import pallas as pl
from jax.experimental.pallas import tpu as pltpu
from jax.experimental.pallas import tpu_sc as plsc

_SC_INFO = plsc.get_sparse_core_info()
NC = _SC_INFO.num_cores
NS = _SC_INFO.num_subcores
NW = NC * NS

DIM = 768
FF = 1536
E = 8
TOP_K = 2
T = 2048
A = T * TOP_K
TILE = 1024
P = A + E * TILE
NT = P // TILE


def _router_body(x_ref, gw_ref, gb_ref, dest_ref, eid_ref, valid_ref,
                 w01_ref):
    xf = x_ref[...]
    gw = gw_ref[...]
    logits = lax.dot_general(xf, gw, (((1,), (1,)), ((), ())),
                             preferred_element_type=jnp.float32)
    logits = logits + gb_ref[...]
    m = jnp.max(logits, axis=1, keepdims=True)
    p = jnp.exp(logits - m)
    probs = p / jnp.sum(p, axis=1, keepdims=True)

    lane = lax.broadcasted_iota(jnp.int32, (T, E), 1)
    m1 = jnp.max(probs, axis=1, keepdims=True)
    i1 = jnp.min(jnp.where(probs >= m1, lane, E), axis=1, keepdims=True)
    probs2 = jnp.where(lane == i1, -jnp.inf, probs)
    m2 = jnp.max(probs2, axis=1, keepdims=True)
    i2 = jnp.min(jnp.where(probs2 >= m2, lane, E), axis=1, keepdims=True)
    sw = m1 + m2 + 1e-6
    w01_ref[...] = jnp.concatenate([m1 / sw, m2 / sw], axis=1)

    e_col = jnp.concatenate([i1, i2], axis=0)
    lane_a = lax.broadcasted_iota(jnp.int32, (A, E), 1)
    oh = (e_col == lane_a).astype(jnp.int32)
    c = oh
    sh = 1
    while sh < A:
        c = c + jnp.concatenate(
            [jnp.zeros((sh, E), jnp.int32), c[:-sh, :]], axis=0)
        sh *= 2
    rank = jnp.sum(oh * c, axis=1, keepdims=True) - 1
    counts = c[A - 1:A, :]
    pad_counts = ((counts + TILE - 1) // TILE) * TILE
    tri = (lax.broadcasted_iota(jnp.int32, (E, E), 0) <
           lax.broadcasted_iota(jnp.int32, (E, E), 1)).astype(jnp.float32)
    base = lax.dot_general(pad_counts.astype(jnp.float32), tri,
                           (((1,), (0,)), ((), ())),
                           preferred_element_type=jnp.float32).astype(jnp.int32)
    dest_ref[...] = jnp.sum(oh * base, axis=1, keepdims=True) + rank
    ts = base // TILE
    ntiles = pad_counts // TILE
    jj = lax.broadcasted_iota(jnp.int32, (NT, E), 0)
    lane_t = lax.broadcasted_iota(jnp.int32, (NT, E), 1)
    eid_ref[...] = jnp.max(
        jnp.where((jj >= ts) & (counts > 0), lane_t, 0),
        axis=1, keepdims=True)
    total_tiles = jnp.sum(ntiles, axis=1, keepdims=True)
    valid_ref[...] = (lax.broadcasted_iota(jnp.int32, (NT, 1), 0) <
                      total_tiles).astype(jnp.int32)


def _router(xf, gate_w, gate_b):
    return pl.pallas_call(
        _router_body,
        out_shape=(
            jax.ShapeDtypeStruct((A, 1), jnp.int32),
            jax.ShapeDtypeStruct((NT, 1), jnp.int32),
            jax.ShapeDtypeStruct((NT, 1), jnp.int32),
            jax.ShapeDtypeStruct((T, 2), jnp.float32),
        ),
    )(xf, gate_w, gate_b.reshape(1, E))


def _ffn_body(eid_ref, valid_ref, xs_ref, w1_ref, b1_ref, w2_ref, b2_ref,
              y_ref):
    @pl.when(valid_ref[pl.program_id(0)] > 0)
    def _():
        h = lax.dot_general(xs_ref[...], w1_ref[0], (((1,), (1,)), ((), ())),
                            preferred_element_type=jnp.float32) + b1_ref[0]
        hh = h[:, :FF]
        g = h[:, FF:]
        act = hh * (0.5 * g * (1.0 + lax.erf(g * 0.7071067811865476)))
        y_ref[...] = lax.dot_general(act, w2_ref[0], (((1,), (1,)), ((), ())),
                                     preferred_element_type=jnp.float32) + b2_ref[0]


def _ffn(xs, tile_eid, tile_valid, fc1_w, fc1_b, fc2_w, fc2_b):
    grid_spec = pltpu.PrefetchScalarGridSpec(
        num_scalar_prefetch=2,
        grid=(NT,),
        in_specs=[
            pl.BlockSpec((TILE, DIM), lambda j, eid, v: (j, 0)),
            pl.BlockSpec((1, 2 * FF, DIM), lambda j, eid, v: (eid[j], 0, 0)),
            pl.BlockSpec((1, 1, 2 * FF), lambda j, eid, v: (eid[j], 0, 0)),
            pl.BlockSpec((1, DIM, FF), lambda j, eid, v: (eid[j], 0, 0)),
            pl.BlockSpec((1, 1, DIM), lambda j, eid, v: (eid[j], 0, 0)),
        ],
        out_specs=pl.BlockSpec((TILE, DIM), lambda j, eid, v: (j, 0)),
    )
    return pl.pallas_call(
        _ffn_body,
        grid_spec=grid_spec,
        out_shape=jax.ShapeDtypeStruct((P, DIM), jnp.float32),
        compiler_params=pltpu.CompilerParams(
            dimension_semantics=("parallel",)),
    )(tile_eid, tile_valid, xs, fc1_w, fc1_b.reshape(E, 1, 2 * FF),
      fc2_w, fc2_b.reshape(E, 1, DIM))


def _combine_body(g0_ref, g1_ref, w01_ref, out_ref):
    w0 = w01_ref[:, 0:1]
    w1 = w01_ref[:, 1:2]
    out_ref[...] = w0 * g0_ref[...] + w1 * g1_ref[...]


def _combine(g0, g1, w01):
    return pl.pallas_call(
        _combine_body,
        out_shape=jax.ShapeDtypeStruct((T, DIM), jnp.float32),
    )(g0, g1, w01)


def _dispatch(xf, dest):
    bpw = A // NW
    mesh = plsc.VectorSubcoreMesh(core_axis_name="c", subcore_axis_name="s")

    @functools.partial(
        pl.kernel, mesh=mesh,
        out_type=jax.ShapeDtypeStruct((P, DIM), jnp.float32),
        scratch_types=[
            pltpu.VMEM((bpw,), jnp.int32),
            pltpu.VMEM((bpw, DIM), jnp.float32),
            pltpu.SemaphoreType.DMA,
        ],
    )
    def k(x_hbm, dest_hbm, xs_hbm, idx_v, rows_v, sem):
        wid = lax.axis_index("s") * NC + lax.axis_index("c")
        base = wid * bpw
        src = lax.rem(base, T)
        pltpu.sync_copy(dest_hbm.at[pl.ds(base, bpw)], idx_v)
        pltpu.sync_copy(x_hbm.at[pl.ds(src, bpw)], rows_v)
        pltpu.async_copy(rows_v, xs_hbm.at[idx_v], sem).wait()

    return k(xf, dest)


def _gather_pair(ys, d0, d1):
    bpw = T // NW
    mesh = plsc.VectorSubcoreMesh(core_axis_name="c", subcore_axis_name="s")

    @functools.partial(
        pl.kernel, mesh=mesh,
        out_type=(
            jax.ShapeDtypeStruct((T, DIM), jnp.float32),
            jax.ShapeDtypeStruct((T, DIM), jnp.float32),
        ),
        scratch_types=[
            pltpu.VMEM((bpw,), jnp.int32),
            pltpu.VMEM((bpw,), jnp.int32),
            pltpu.VMEM((bpw, DIM), jnp.float32),
            pltpu.VMEM((bpw, DIM), jnp.float32),
            pltpu.SemaphoreType.DMA,
        ],
    )
    def k(ys_hbm, d0_hbm, d1_hbm, g0_hbm, g1_hbm, i0_v, i1_v, r0_v, r1_v, sem):
        wid = lax.axis_index("s") * NC + lax.axis_index("c")
        base = wid * bpw
        pltpu.sync_copy(d0_hbm.at[pl.ds(base, bpw)], i0_v)
        pltpu.sync_copy(d1_hbm.at[pl.ds(base, bpw)], i1_v)
        c0 = pltpu.async_copy(ys_hbm.at[i0_v], r0_v, sem)
        c1 = pltpu.async_copy(ys_hbm.at[i1_v], r1_v, sem)
        c0.wait()
        c1.wait()
        pltpu.sync_copy(r0_v, g0_hbm.at[pl.ds(base, bpw)])
        pltpu.sync_copy(r1_v, g1_hbm.at[pl.ds(base, bpw)])

    return k(ys, d0, d1)


def kernel(x, gate_w, gate_b, fc1_w, fc1_b, fc2_w, fc2_b):
    xf = x.reshape(T, DIM)
    dest, tile_eid, tile_valid, w01 = _router(xf, gate_w, gate_b)
    dest_f = dest.reshape(A)
    xs = _dispatch(xf, dest_f)
    ys = _ffn(xs, tile_eid.reshape(NT), tile_valid.reshape(NT),
              fc1_w, fc1_b, fc2_w, fc2_b)
    g0, g1 = _gather_pair(ys, dest_f[:T], dest_f[T:])
    out = _combine(g0, g1, w01)
    return out.reshape(1, T, DIM)


# speedup vs baseline: 1.0592x; 1.0592x over previous
import functools

import jax
import jax.numpy as jnp
from jax import lax
from jax.experimental import pallas as pl
from jax.experimental.pallas import tpu as pltpu
from jax.experimental.pallas import tpu_sc as plsc

_SC_INFO = plsc.get_sparse_core_info()
NC = _SC_INFO.num_cores
NS = _SC_INFO.num_subcores
NW = NC * NS

DIM = 768
FF = 1536
E = 8
TOP_K = 2
T = 2048
A = T * TOP_K
TILE = 512
P = A + E * TILE
NT = P // TILE


def _router_body(x_ref, gw_ref, gb_ref, dest_ref, eid_ref, valid_ref,
                 w01_ref):
    xf = x_ref[...]
    gw = gw_ref[...]
    logits = lax.dot_general(xf, gw, (((1,), (1,)), ((), ())),
                             preferred_element_type=jnp.float32)
    logits = logits + gb_ref[...]
    m = jnp.max(logits, axis=1, keepdims=True)
    p = jnp.exp(logits - m)
    probs = p / jnp.sum(p, axis=1, keepdims=True)

    lane = lax.broadcasted_iota(jnp.int32, (T, E), 1)
    m1 = jnp.max(probs, axis=1, keepdims=True)
    i1 = jnp.min(jnp.where(probs >= m1, lane, E), axis=1, keepdims=True)
    probs2 = jnp.where(lane == i1, -jnp.inf, probs)
    m2 = jnp.max(probs2, axis=1, keepdims=True)
    i2 = jnp.min(jnp.where(probs2 >= m2, lane, E), axis=1, keepdims=True)
    sw = m1 + m2 + 1e-6
    w01_ref[...] = jnp.concatenate([m1 / sw, m2 / sw], axis=1)

    e_col = jnp.concatenate([i1, i2], axis=0)
    lane_a = lax.broadcasted_iota(jnp.int32, (A, E), 1)
    oh = (e_col == lane_a).astype(jnp.int32)
    c = oh
    sh = 1
    while sh < A:
        c = c + jnp.concatenate(
            [jnp.zeros((sh, E), jnp.int32), c[:-sh, :]], axis=0)
        sh *= 2
    rank = jnp.sum(oh * c, axis=1, keepdims=True) - 1
    counts = c[A - 1:A, :]
    pad_counts = ((counts + TILE - 1) // TILE) * TILE
    tri = (lax.broadcasted_iota(jnp.int32, (E, E), 0) <
           lax.broadcasted_iota(jnp.int32, (E, E), 1)).astype(jnp.float32)
    base = lax.dot_general(pad_counts.astype(jnp.float32), tri,
                           (((1,), (0,)), ((), ())),
                           preferred_element_type=jnp.float32).astype(jnp.int32)
    dest_ref[...] = jnp.sum(oh * base, axis=1, keepdims=True) + rank
    ts = base // TILE
    ntiles = pad_counts // TILE
    jj = lax.broadcasted_iota(jnp.int32, (NT, E), 0)
    lane_t = lax.broadcasted_iota(jnp.int32, (NT, E), 1)
    eid_ref[...] = jnp.max(
        jnp.where((jj >= ts) & (counts > 0), lane_t, 0),
        axis=1, keepdims=True)
    total_tiles = jnp.sum(ntiles, axis=1, keepdims=True)
    valid_ref[...] = (lax.broadcasted_iota(jnp.int32, (NT, 1), 0) <
                      total_tiles).astype(jnp.int32)


def _router(xf, gate_w, gate_b):
    return pl.pallas_call(
        _router_body,
        out_shape=(
            jax.ShapeDtypeStruct((A, 1), jnp.int32),
            jax.ShapeDtypeStruct((NT, 1), jnp.int32),
            jax.ShapeDtypeStruct((NT, 1), jnp.int32),
            jax.ShapeDtypeStruct((T, 2), jnp.float32),
        ),
    )(xf, gate_w, gate_b.reshape(1, E))


def _ffn_body(eid_ref, valid_ref, xs_ref, w1_ref, b1_ref, w2_ref, b2_ref,
              y_ref):
    @pl.when(valid_ref[pl.program_id(0)] > 0)
    def _():
        h = lax.dot_general(xs_ref[...], w1_ref[0], (((1,), (1,)), ((), ())),
                            preferred_element_type=jnp.float32) + b1_ref[0]
        hh = h[:, :FF]
        g = h[:, FF:]
        act = hh * (0.5 * g * (1.0 + lax.erf(g * 0.7071067811865476)))
        y_ref[...] = lax.dot_general(act, w2_ref[0], (((1,), (1,)), ((), ())),
                                     preferred_element_type=jnp.float32) + b2_ref[0]


def _ffn(xs, tile_eid, tile_valid, fc1_w, fc1_b, fc2_w, fc2_b):
    grid_spec = pltpu.PrefetchScalarGridSpec(
        num_scalar_prefetch=2,
        grid=(NT,),
        in_specs=[
            pl.BlockSpec((TILE, DIM), lambda j, eid, v: (j, 0)),
            pl.BlockSpec((1, 2 * FF, DIM), lambda j, eid, v: (eid[j], 0, 0)),
            pl.BlockSpec((1, 1, 2 * FF), lambda j, eid, v: (eid[j], 0, 0)),
            pl.BlockSpec((1, DIM, FF), lambda j, eid, v: (eid[j], 0, 0)),
            pl.BlockSpec((1, 1, DIM), lambda j, eid, v: (eid[j], 0, 0)),
        ],
        out_specs=pl.BlockSpec((TILE, DIM), lambda j, eid, v: (j, 0)),
    )
    return pl.pallas_call(
        _ffn_body,
        grid_spec=grid_spec,
        out_shape=jax.ShapeDtypeStruct((P, DIM), jnp.float32),
        compiler_params=pltpu.CompilerParams(
            dimension_semantics=("parallel",)),
    )(tile_eid, tile_valid, xs, fc1_w, fc1_b.reshape(E, 1, 2 * FF),
      fc2_w, fc2_b.reshape(E, 1, DIM))


def _combine_body(g0_ref, g1_ref, w01_ref, out_ref):
    w0 = w01_ref[:, 0:1]
    w1 = w01_ref[:, 1:2]
    out_ref[...] = w0 * g0_ref[...] + w1 * g1_ref[...]


def _combine(g0, g1, w01):
    return pl.pallas_call(
        _combine_body,
        out_shape=jax.ShapeDtypeStruct((T, DIM), jnp.float32),
    )(g0, g1, w01)


def _dispatch(xf, dest):
    bpw = A // NW
    mesh = plsc.VectorSubcoreMesh(core_axis_name="c", subcore_axis_name="s")

    @functools.partial(
        pl.kernel, mesh=mesh,
        out_type=jax.ShapeDtypeStruct((P, DIM), jnp.float32),
        scratch_types=[
            pltpu.VMEM((bpw,), jnp.int32),
            pltpu.VMEM((bpw, DIM), jnp.float32),
            pltpu.SemaphoreType.DMA,
        ],
    )
    def k(x_hbm, dest_hbm, xs_hbm, idx_v, rows_v, sem):
        wid = lax.axis_index("s") * NC + lax.axis_index("c")
        base = wid * bpw
        src = lax.rem(base, T)
        pltpu.sync_copy(dest_hbm.at[pl.ds(base, bpw)], idx_v)
        pltpu.sync_copy(x_hbm.at[pl.ds(src, bpw)], rows_v)
        pltpu.async_copy(rows_v, xs_hbm.at[idx_v], sem).wait()

    return k(xf, dest)


def _gather_pair(ys, dest):
    bpw = T // NW
    mesh = plsc.VectorSubcoreMesh(core_axis_name="c", subcore_axis_name="s")

    @functools.partial(
        pl.kernel, mesh=mesh,
        out_type=(
            jax.ShapeDtypeStruct((T, DIM), jnp.float32),
            jax.ShapeDtypeStruct((T, DIM), jnp.float32),
        ),
        scratch_types=[
            pltpu.VMEM((bpw,), jnp.int32),
            pltpu.VMEM((bpw,), jnp.int32),
            pltpu.VMEM((bpw, DIM), jnp.float32),
            pltpu.VMEM((bpw, DIM), jnp.float32),
            pltpu.SemaphoreType.DMA,
        ],
    )
    def k(ys_hbm, dest_hbm, g0_hbm, g1_hbm, i0_v, i1_v, r0_v, r1_v, sem):
        wid = lax.axis_index("s") * NC + lax.axis_index("c")
        base = wid * bpw
        pltpu.sync_copy(dest_hbm.at[pl.ds(base, bpw)], i0_v)
        pltpu.sync_copy(dest_hbm.at[pl.ds(T + base, bpw)], i1_v)
        c0 = pltpu.async_copy(ys_hbm.at[i0_v], r0_v, sem)
        c1 = pltpu.async_copy(ys_hbm.at[i1_v], r1_v, sem)
        c0.wait()
        c1.wait()
        pltpu.sync_copy(r0_v, g0_hbm.at[pl.ds(base, bpw)])
        pltpu.sync_copy(r1_v, g1_hbm.at[pl.ds(base, bpw)])

    return k(ys, dest)


def kernel(x, gate_w, gate_b, fc1_w, fc1_b, fc2_w, fc2_b):
    xf = x.reshape(T, DIM)
    dest, tile_eid, tile_valid, w01 = _router(xf, gate_w, gate_b)
    dest_f = dest.reshape(A)
    xs = _dispatch(xf, dest_f)
    ys = _ffn(xs, tile_eid.reshape(NT), tile_valid.reshape(NT),
              fc1_w, fc1_b, fc2_w, fc2_b)
    g0, g1 = _gather_pair(ys, dest_f)
    out = _combine(g0, g1, w01)
    return out.reshape(1, T, DIM)
